# Initial kernel scaffold; baseline (speedup 1.0000x reference)
import jax
import jax.numpy as jnp
from jax.experimental import pallas as pl


def kernel(input_ids, table):
    raise NotImplementedError("write your pallas kernel here")


# speedup vs baseline: 2.6947x; 2.6947x over previous
import functools

import jax
import jax.numpy as jnp
from jax import lax
from jax.experimental import pallas as pl
from jax.experimental.pallas import tpu as pltpu
from jax.experimental.pallas import tpu_sc as plsc

NC = 2
NS = 16
LANES = 16
NW = NC * NS


@functools.lru_cache(maxsize=None)
def _make_kernel(B, L, V, D):
    UNK = V - 2
    PAD = V - 1
    DG = D // LANES
    ROWS_W = B // NW
    R = 4
    TOK = R * L
    NCHUNK = ROWS_W // R
    MGROUPS = list(range(0, TOK - 15, 16))
    if MGROUPS[-1] != TOK - 16:
        MGROUPS.append(TOK - 16)
    SPLITS = []
    off = 0
    while off < TOK:
        n = min(128, TOK - off)
        SPLITS.append((off, n))
        off += n

    mesh = plsc.VectorSubcoreMesh(
        core_axis_name="c", subcore_axis_name="s",
        num_cores=NC, num_subcores=NS)

    @functools.partial(
        pl.kernel,
        out_type=jax.ShapeDtypeStruct((B * L, D), jnp.float32),
        mesh=mesh,
        compiler_params=pltpu.CompilerParams(needs_layout_passes=False),
        scratch_types=[
            pltpu.VMEM((TOK,), jnp.int32),
            pltpu.VMEM((TOK,), jnp.int32),
            pltpu.VMEM((TOK + 24,), jnp.float32),
            pltpu.VMEM((TOK, D), jnp.float32),
            pltpu.SemaphoreType.DMA,
        ],
    )
    def embed_kernel(ids_hbm, table_hbm, out_hbm, ids_v, idx_v, mask_v,
                     emb_v, sem):
        wid = lax.axis_index("s") * NC + lax.axis_index("c")
        iota = lax.iota(jnp.int32, LANES)

        def chunk_body(ci, carry):
            base_tok = (wid * ROWS_W + ci * R) * L
            pltpu.sync_copy(ids_hbm.at[pl.ds(base_tok, TOK)], ids_v)
            for off in MGROUPS:
                idv = ids_v[pl.ds(off, LANES)]
                valid = (idv != UNK) & (idv != PAD)
                idx_v[pl.ds(off, LANES)] = jnp.where(valid, idv, PAD)
                mask_v[pl.ds(off, LANES)] = jnp.where(valid, 1.0, 0.0).astype(
                    jnp.float32)
            copies = [
                pltpu.async_copy(table_hbm.at[idx_v.at[pl.ds(o, n)]],
                                 emb_v.at[pl.ds(o, n)], sem)
                for o, n in SPLITS
            ]
            for cp in copies:
                cp.wait()

            for r in range(R):
                tbase = r * L
                g0 = plsc.load_gather(mask_v, [iota + tbase])
                g1 = plsc.load_gather(mask_v, [iota + (tbase + 16)])
                g2 = plsc.load_gather(mask_v, [iota + (tbase + 32)])
                g3 = plsc.load_gather(mask_v, [iota + (tbase + 48)])
                cnt_l = g0 + g1 + g2 + jnp.where(iota < L - 48, g3, 0.0)
                cnt = jnp.sum(cnt_l)

                def sum_body(t, accs):
                    row = tbase + t
                    return tuple(
                        accs[d] + emb_v[row, pl.ds(LANES * d, LANES)]
                        for d in range(DG))

                accs = lax.fori_loop(
                    0, L, sum_body,
                    tuple(jnp.zeros((LANES,), jnp.float32)
                          for _ in range(DG)))
                denom = lax.broadcast(cnt, (LANES,)) + 1e-8
                avgs = tuple(a / denom for a in accs)

                def out_body(t, carry2):
                    row = tbase + t
                    w = 1.0 - plsc.load_gather(
                        mask_v, [lax.broadcast(row, (LANES,))])
                    for d in range(DG):
                        sl = pl.ds(LANES * d, LANES)
                        emb_v[row, sl] = emb_v[row, sl] + avgs[d] * w
                    return carry2

                lax.fori_loop(0, L, out_body, 0)

            pltpu.sync_copy(emb_v, out_hbm.at[pl.ds(base_tok, TOK)])
            return carry

        lax.fori_loop(0, NCHUNK, chunk_body, 0)

    return embed_kernel


def kernel(input_ids, table):
    B, L = input_ids.shape
    V, D = table.shape
    k = _make_kernel(B, L, V, D)
    out = k(input_ids.reshape(-1).astype(jnp.int32), table)
    return out.reshape(B, L, D)


# speedup vs baseline: 2.9790x; 1.1055x over previous
import functools

import jax
import jax.numpy as jnp
from jax import lax
from jax.experimental import pallas as pl
from jax.experimental.pallas import tpu as pltpu
from jax.experimental.pallas import tpu_sc as plsc

NC = 2
NS = 16
LANES = 16
NW = NC * NS


@functools.lru_cache(maxsize=None)
def _make_kernel(B, L, V, D):
    UNK = V - 2
    PAD = V - 1
    DG = D // LANES
    ROWS_W = B // NW
    R = 8
    TOK = R * L
    NCHUNK = ROWS_W // R
    NPAIR = NCHUNK // 2
    assert TOK % LANES == 0
    MGROUPS = list(range(0, TOK, LANES))
    SPLITS = []
    off = 0
    while off < TOK:
        n = min(128, TOK - off)
        SPLITS.append((off, n))
        off += n

    mesh = plsc.VectorSubcoreMesh(
        core_axis_name="c", subcore_axis_name="s",
        num_cores=NC, num_subcores=NS)

    @functools.partial(
        pl.kernel,
        out_type=jax.ShapeDtypeStruct((B * L, D), jnp.float32),
        mesh=mesh,
        compiler_params=pltpu.CompilerParams(needs_layout_passes=False),
        scratch_types=[
            pltpu.VMEM((2 * TOK,), jnp.int32),
            pltpu.VMEM((TOK,), jnp.int32),
            pltpu.VMEM((TOK,), jnp.int32),
            pltpu.VMEM((TOK + 24,), jnp.float32),
            pltpu.VMEM((TOK + 24,), jnp.float32),
            pltpu.VMEM((TOK, D), jnp.float32),
            pltpu.VMEM((TOK, D), jnp.float32),
            pltpu.SemaphoreType.DMA,
            pltpu.SemaphoreType.DMA,
            pltpu.SemaphoreType.DMA,
            pltpu.SemaphoreType.DMA,
        ],
    )
    def embed_kernel(ids_hbm, table_hbm, out_hbm, ids_v,
                     idx_v0, idx_v1, mask_v0, mask_v1, emb_v0, emb_v1,
                     sem_g0, sem_g1, sem_o0, sem_o1):
        wid = lax.axis_index("s") * NC + lax.axis_index("c")
        iota = lax.iota(jnp.int32, LANES)

        def build(ids_off, idx_v, mask_v):
            for off in MGROUPS:
                idv = ids_v[pl.ds(ids_off + off, LANES)]
                valid = (idv != UNK) & (idv != PAD)
                idx_v[pl.ds(off, LANES)] = jnp.where(valid, idv, PAD)
                mask_v[pl.ds(off, LANES)] = jnp.where(valid, 1.0, 0.0).astype(
                    jnp.float32)

        def start_gather(idx_v, emb_v, sem):
            return [
                pltpu.async_copy(table_hbm.at[idx_v.at[pl.ds(o, n)]],
                                 emb_v.at[pl.ds(o, n)], sem)
                for o, n in SPLITS
            ]

        def compute(mask_v, emb_v):
            for r in range(R):
                tbase = r * L
                g0 = plsc.load_gather(mask_v, [iota + tbase])
                g1 = plsc.load_gather(mask_v, [iota + (tbase + 16)])
                g2 = plsc.load_gather(mask_v, [iota + (tbase + 32)])
                g3 = plsc.load_gather(mask_v, [iota + (tbase + 48)])
                cnt_l = g0 + g1 + g2 + jnp.where(iota < L - 48, g3, 0.0)
                cnt = jnp.sum(cnt_l)

                def sum_body(t, accs):
                    row = tbase + t
                    return tuple(
                        accs[d] + emb_v[row, pl.ds(LANES * d, LANES)]
                        for d in range(DG))

                accs = lax.fori_loop(
                    0, L, sum_body,
                    tuple(jnp.zeros((LANES,), jnp.float32)
                          for _ in range(DG)))
                denom = lax.broadcast(cnt, (LANES,)) + 1e-8
                avgs = tuple(a / denom for a in accs)

                def out_body(t, carry2):
                    row = tbase + t
                    w = 1.0 - plsc.load_gather(
                        mask_v, [lax.broadcast(row, (LANES,))])
                    for d in range(DG):
                        sl = pl.ds(LANES * d, LANES)
                        emb_v[row, sl] = emb_v[row, sl] + avgs[d] * w
                    return carry2

                lax.fori_loop(0, L, out_body, 0)

        def pair_body(p, carry):
            base_tok = (wid * ROWS_W + p * 2 * R) * L
            pltpu.sync_copy(ids_hbm.at[pl.ds(base_tok, 2 * TOK)], ids_v)
            build(0, idx_v0, mask_v0)
            cps0 = start_gather(idx_v0, emb_v0, sem_g0)
            build(TOK, idx_v1, mask_v1)
            cps1 = start_gather(idx_v1, emb_v1, sem_g1)
            for cp in cps0:
                cp.wait()
            compute(mask_v0, emb_v0)
            out0 = pltpu.async_copy(
                emb_v0, out_hbm.at[pl.ds(base_tok, TOK)], sem_o0)
            for cp in cps1:
                cp.wait()
            compute(mask_v1, emb_v1)
            out1 = pltpu.async_copy(
                emb_v1, out_hbm.at[pl.ds(base_tok + TOK, TOK)], sem_o1)
            out0.wait()
            out1.wait()
            return carry

        lax.fori_loop(0, NPAIR, pair_body, 0)

    return embed_kernel


def kernel(input_ids, table):
    B, L = input_ids.shape
    V, D = table.shape
    k = _make_kernel(B, L, V, D)
    out = k(input_ids.reshape(-1).astype(jnp.int32), table)
    return out.reshape(B, L, D)


# speedup vs baseline: 3.8692x; 1.2988x over previous
import functools

import jax
import jax.numpy as jnp
from jax import lax
from jax.experimental import pallas as pl
from jax.experimental.pallas import tpu as pltpu
from jax.experimental.pallas import tpu_sc as plsc

NC = 2
NS = 16
LANES = 16
NW = NC * NS


@functools.lru_cache(maxsize=None)
def _make_kernel(B, L, V, D):
    UNK = V - 2
    PAD = V - 1
    DG = D // LANES
    ROWS_W = B // NW
    R = 8
    TOK = R * L
    NCHUNK = ROWS_W // R
    NPAIR = NCHUNK // 2
    assert TOK % LANES == 0
    MGROUPS = list(range(0, TOK, LANES))
    SPLITS = []
    off = 0
    while off < TOK:
        n = min(128, TOK - off)
        SPLITS.append((off, n))
        off += n

    mesh = plsc.VectorSubcoreMesh(
        core_axis_name="c", subcore_axis_name="s",
        num_cores=NC, num_subcores=NS)

    @functools.partial(
        pl.kernel,
        out_type=jax.ShapeDtypeStruct((B * L, D), jnp.float32),
        mesh=mesh,
        compiler_params=pltpu.CompilerParams(needs_layout_passes=False),
        scratch_types=[
            pltpu.VMEM((2 * TOK,), jnp.int32),
            pltpu.VMEM((TOK,), jnp.int32),
            pltpu.VMEM((TOK,), jnp.int32),
            pltpu.VMEM((TOK + 24,), jnp.float32),
            pltpu.VMEM((TOK + 24,), jnp.float32),
            pltpu.VMEM((TOK, D), jnp.float32),
            pltpu.VMEM((TOK, D), jnp.float32),
            pltpu.SemaphoreType.DMA,
            pltpu.SemaphoreType.DMA,
            pltpu.SemaphoreType.DMA,
            pltpu.SemaphoreType.DMA,
        ],
    )
    def embed_kernel(ids_hbm, table_hbm, out_hbm, ids_v,
                     idx_v0, idx_v1, mask_v0, mask_v1, emb_v0, emb_v1,
                     sem_g0, sem_g1, sem_o0, sem_o1):
        wid = lax.axis_index("s") * NC + lax.axis_index("c")
        iota = lax.iota(jnp.int32, LANES)

        def build(ids_off, idx_v, mask_v):
            for off in MGROUPS:
                idv = ids_v[pl.ds(ids_off + off, LANES)]
                valid = (idv != UNK) & (idv != PAD)
                idx_v[pl.ds(off, LANES)] = jnp.where(valid, idv, PAD)
                mask_v[pl.ds(off, LANES)] = jnp.where(valid, 1.0, 0.0).astype(
                    jnp.float32)

        def start_gather(idx_v, emb_v, sem):
            return [
                pltpu.async_copy(table_hbm.at[idx_v.at[pl.ds(o, n)]],
                                 emb_v.at[pl.ds(o, n)], sem)
                for o, n in SPLITS
            ]

        def fix_row(mask_v, emb_v, tbase, cnt):
            def sum_body(t, accs):
                row = tbase + t
                return tuple(
                    accs[d] + emb_v[row, pl.ds(LANES * d, LANES)]
                    for d in range(DG))

            accs = lax.fori_loop(
                0, L, sum_body,
                tuple(jnp.zeros((LANES,), jnp.float32) for _ in range(DG)))
            denom = lax.broadcast(cnt, (LANES,)) + 1e-8
            avgs = tuple(a / denom for a in accs)

            def out_body(t, carry2):
                row = tbase + t
                w = 1.0 - plsc.load_gather(
                    mask_v, [lax.broadcast(row, (LANES,))])
                for d in range(DG):
                    sl = pl.ds(LANES * d, LANES)
                    emb_v[row, sl] = emb_v[row, sl] + avgs[d] * w
                return carry2

            lax.fori_loop(0, L, out_body, 0)

        def compute(mask_v, emb_v):
            for r in range(R):
                tbase = r * L
                g0 = plsc.load_gather(mask_v, [iota + tbase])
                g1 = plsc.load_gather(mask_v, [iota + (tbase + 16)])
                g2 = plsc.load_gather(mask_v, [iota + (tbase + 32)])
                g3 = plsc.load_gather(mask_v, [iota + (tbase + 48)])
                cnt_l = g0 + g1 + g2 + jnp.where(iota < L - 48, g3, 0.0)
                cnt = jnp.sum(cnt_l)
                pl.when(cnt != float(L))(
                    lambda m=mask_v, e=emb_v, tb=tbase, c=cnt:
                        fix_row(m, e, tb, c))

        def drain_out(emb_v, sem):
            pltpu.make_async_copy(
                emb_v, out_hbm.at[pl.ds(0, TOK)], sem).wait()

        def pair_body(p, carry):
            base_tok = (wid * ROWS_W + p * 2 * R) * L
            pltpu.sync_copy(ids_hbm.at[pl.ds(base_tok, 2 * TOK)], ids_v)
            build(0, idx_v0, mask_v0)
            pl.when(p > 0)(lambda: drain_out(emb_v0, sem_o0))
            cps0 = start_gather(idx_v0, emb_v0, sem_g0)
            build(TOK, idx_v1, mask_v1)
            pl.when(p > 0)(lambda: drain_out(emb_v1, sem_o1))
            cps1 = start_gather(idx_v1, emb_v1, sem_g1)
            for cp in cps0:
                cp.wait()
            compute(mask_v0, emb_v0)
            pltpu.async_copy(
                emb_v0, out_hbm.at[pl.ds(base_tok, TOK)], sem_o0)
            for cp in cps1:
                cp.wait()
            compute(mask_v1, emb_v1)
            pltpu.async_copy(
                emb_v1, out_hbm.at[pl.ds(base_tok + TOK, TOK)], sem_o1)
            return carry

        lax.fori_loop(0, NPAIR, pair_body, 0)
        drain_out(emb_v0, sem_o0)
        drain_out(emb_v1, sem_o1)

    return embed_kernel


def kernel(input_ids, table):
    B, L = input_ids.shape
    V, D = table.shape
    k = _make_kernel(B, L, V, D)
    out = k(input_ids.reshape(-1).astype(jnp.int32), table)
    return out.reshape(B, L, D)


# speedup vs baseline: 6.7498x; 1.7445x over previous
import functools

import jax
import jax.numpy as jnp
from jax import lax
from jax.experimental import pallas as pl
from jax.experimental.pallas import tpu as pltpu
from jax.experimental.pallas import tpu_sc as plsc

NC = 2
NS = 16
LANES = 16
NW = NC * NS


@functools.lru_cache(maxsize=None)
def _make_kernel(B, L, V, D):
    UNK = V - 2
    PAD = V - 1
    DG = D // LANES
    ROWS_W = B // NW
    R = 8
    TOK = R * L
    NCHUNK = ROWS_W // R
    NPAIR = NCHUNK // 2
    assert TOK % LANES == 0
    MGROUPS = list(range(0, TOK, LANES))
    SPLITS = []
    off = 0
    while off < TOK:
        n = min(128, TOK - off)
        SPLITS.append((off, n))
        off += n

    mesh = plsc.VectorSubcoreMesh(
        core_axis_name="c", subcore_axis_name="s",
        num_cores=NC, num_subcores=NS)

    @functools.partial(
        pl.kernel,
        out_type=jax.ShapeDtypeStruct((B, L, D), jnp.float32),
        mesh=mesh,
        compiler_params=pltpu.CompilerParams(needs_layout_passes=False),
        scratch_types=[
            pltpu.VMEM((2 * TOK,), jnp.int32),
            pltpu.VMEM((TOK,), jnp.int32),
            pltpu.VMEM((TOK,), jnp.int32),
            pltpu.VMEM((TOK + 24,), jnp.float32),
            pltpu.VMEM((TOK + 24,), jnp.float32),
            pltpu.VMEM((TOK, D), jnp.float32),
            pltpu.VMEM((TOK, D), jnp.float32),
            pltpu.SemaphoreType.DMA,
            pltpu.SemaphoreType.DMA,
            pltpu.SemaphoreType.DMA,
            pltpu.SemaphoreType.DMA,
        ],
    )
    def embed_kernel(ids_hbm, table_hbm, out_hbm, ids_v,
                     idx_v0, idx_v1, mask_v0, mask_v1, emb_v0, emb_v1,
                     sem_g0, sem_g1, sem_o0, sem_o1):
        wid = lax.axis_index("s") * NC + lax.axis_index("c")
        iota = lax.iota(jnp.int32, LANES)

        def build(ids_off, idx_v, mask_v):
            for off in MGROUPS:
                idv = ids_v[pl.ds(ids_off + off, LANES)]
                valid = (idv != UNK) & (idv != PAD)
                idx_v[pl.ds(off, LANES)] = jnp.where(valid, idv, PAD)
                mask_v[pl.ds(off, LANES)] = jnp.where(valid, 1.0, 0.0).astype(
                    jnp.float32)

        def start_gather(idx_v, emb_v, sem):
            return [
                pltpu.async_copy(table_hbm.at[idx_v.at[pl.ds(o, n)]],
                                 emb_v.at[pl.ds(o, n)], sem)
                for o, n in SPLITS
            ]

        def fix_row(mask_v, emb_v, tbase, cnt):
            def sum_body(t, accs):
                row = tbase + t
                return tuple(
                    accs[d] + emb_v[row, pl.ds(LANES * d, LANES)]
                    for d in range(DG))

            accs = lax.fori_loop(
                0, L, sum_body,
                tuple(jnp.zeros((LANES,), jnp.float32) for _ in range(DG)))
            denom = lax.broadcast(cnt, (LANES,)) + 1e-8
            avgs = tuple(a / denom for a in accs)

            def out_body(t, carry2):
                row = tbase + t
                w = 1.0 - plsc.load_gather(
                    mask_v, [lax.broadcast(row, (LANES,))])
                for d in range(DG):
                    sl = pl.ds(LANES * d, LANES)
                    emb_v[row, sl] = emb_v[row, sl] + avgs[d] * w
                return carry2

            lax.fori_loop(0, L, out_body, 0)

        def compute(mask_v, emb_v):
            for r in range(R):
                tbase = r * L
                g0 = plsc.load_gather(mask_v, [iota + tbase])
                g1 = plsc.load_gather(mask_v, [iota + (tbase + 16)])
                g2 = plsc.load_gather(mask_v, [iota + (tbase + 32)])
                g3 = plsc.load_gather(mask_v, [iota + (tbase + 48)])
                cnt_l = g0 + g1 + g2 + jnp.where(iota < L - 48, g3, 0.0)
                cnt = jnp.sum(cnt_l)
                pl.when(cnt != float(L))(
                    lambda m=mask_v, e=emb_v, tb=tbase, c=cnt:
                        fix_row(m, e, tb, c))

        def start_out(emb_v, base_row, sem):
            for r in range(R):
                pltpu.async_copy(emb_v.at[pl.ds(r * L, L)],
                                 out_hbm.at[base_row + r], sem)

        def drain_out(emb_v, sem):
            for r in range(R):
                pltpu.make_async_copy(emb_v.at[pl.ds(r * L, L)],
                                      out_hbm.at[0], sem).wait()

        def pair_body(p, carry):
            base_row = wid * ROWS_W + p * 2 * R
            base_tok = base_row * L
            pltpu.sync_copy(ids_hbm.at[pl.ds(base_tok, 2 * TOK)], ids_v)
            build(0, idx_v0, mask_v0)
            pl.when(p > 0)(lambda: drain_out(emb_v0, sem_o0))
            cps0 = start_gather(idx_v0, emb_v0, sem_g0)
            build(TOK, idx_v1, mask_v1)
            pl.when(p > 0)(lambda: drain_out(emb_v1, sem_o1))
            cps1 = start_gather(idx_v1, emb_v1, sem_g1)
            for cp in cps0:
                cp.wait()
            compute(mask_v0, emb_v0)
            start_out(emb_v0, base_row, sem_o0)
            for cp in cps1:
                cp.wait()
            compute(mask_v1, emb_v1)
            start_out(emb_v1, base_row + R, sem_o1)
            return carry

        lax.fori_loop(0, NPAIR, pair_body, 0)
        drain_out(emb_v0, sem_o0)
        drain_out(emb_v1, sem_o1)

    return embed_kernel


def kernel(input_ids, table):
    B, L = input_ids.shape
    V, D = table.shape
    k = _make_kernel(B, L, V, D)
    return k(input_ids.reshape(-1).astype(jnp.int32), table)


# speedup vs baseline: 11.7291x; 1.7377x over previous
import functools

import jax
import jax.numpy as jnp
from jax import lax
from jax.experimental import pallas as pl
from jax.experimental.pallas import tpu as pltpu
from jax.experimental.pallas import tpu_sc as plsc

NC = 2
NS = 16
LANES = 16
NW = NC * NS


@functools.lru_cache(maxsize=None)
def _make_kernel(B, L, V, D):
    UNK = V - 2
    PAD = V - 1
    DG = D // LANES
    ROWS_W = B // NW
    R = 8
    RS = R.bit_length() - 1
    TOK = R * L
    NCHUNK = ROWS_W // R
    NPAIR = NCHUNK // 2
    assert TOK % LANES == 0
    UNIT = 80
    NU = TOK // UNIT
    assert NU * UNIT == TOK and UNIT % LANES == 0 and UNIT <= 128

    mesh = plsc.VectorSubcoreMesh(
        core_axis_name="c", subcore_axis_name="s",
        num_cores=NC, num_subcores=NS)

    @functools.partial(
        pl.kernel,
        out_type=jax.ShapeDtypeStruct((B * L, D), jnp.float32),
        mesh=mesh,
        compiler_params=pltpu.CompilerParams(needs_layout_passes=False),
        scratch_types=[
            pltpu.VMEM((2 * TOK,), jnp.int32),
            pltpu.VMEM((TOK,), jnp.int32),
            pltpu.VMEM((TOK,), jnp.int32),
            pltpu.VMEM((TOK + 112,), jnp.float32),
            pltpu.VMEM((TOK + 112,), jnp.float32),
            pltpu.VMEM((TOK, D), jnp.float32),
            pltpu.VMEM((TOK, D), jnp.float32),
            pltpu.VMEM((TOK,), jnp.int32),
            pltpu.VMEM((NU, UNIT), jnp.int32),
            pltpu.VMEM((NU, UNIT), jnp.int32),
            pltpu.SemaphoreType.DMA((NU,)),
            pltpu.SemaphoreType.DMA((NU,)),
            pltpu.SemaphoreType.DMA((NU,)),
            pltpu.SemaphoreType.DMA((NU,)),
            pltpu.SemaphoreType.DMA,
        ],
    )
    def embed_kernel(ids_hbm, table_hbm, out_hbm, ids_v,
                     idx_v0, idx_v1, mask_v0, mask_v1, emb_v0, emb_v1,
                     pat_v, sidx_v0, sidx_v1,
                     sem_g0, sem_g1, sem_s0, sem_s1, sem_i):
        wid = lax.axis_index("s") * NC + lax.axis_index("c")
        iota = lax.iota(jnp.int32, LANES)

        def build(ids_off, idx_v, mask_v):
            for j0 in range(0, TOK, LANES):
                jv = iota + j0
                rv = jv & (R - 1)
                lv = lax.shift_right_logical(jv, RS)
                pos = rv * L + lv + ids_off
                idv = plsc.load_gather(ids_v, [pos])
                valid = (idv != UNK) & (idv != PAD)
                idx_v[pl.ds(j0, LANES)] = jnp.where(valid, idv, PAD)
                mask_v[pl.ds(j0, LANES)] = jnp.where(valid, 1.0, 0.0).astype(
                    jnp.float32)

        def row_counts(mask_v):
            cnts = []
            for r in range(R):
                g0 = plsc.load_gather(mask_v, [iota * R + r])
                g1 = plsc.load_gather(mask_v, [(iota + 16) * R + r])
                g2 = plsc.load_gather(mask_v, [(iota + 32) * R + r])
                g3 = plsc.load_gather(mask_v, [(iota + 48) * R + r])
                cnt_l = g0 + g1 + g2 + jnp.where(iota < L - 48, g3, 0.0)
                cnts.append(jnp.sum(cnt_l))
            return cnts

        def fix_row(mask_v, emb_v, r, cnt):
            def sum_body(t, accs):
                row = t * R + r
                return tuple(
                    accs[d] + emb_v[row, pl.ds(LANES * d, LANES)]
                    for d in range(DG))

            accs = lax.fori_loop(
                0, L, sum_body,
                tuple(jnp.zeros((LANES,), jnp.float32) for _ in range(DG)))
            denom = lax.broadcast(cnt, (LANES,)) + 1e-8
            avgs = tuple(a / denom for a in accs)

            def out_body(t, carry2):
                row = t * R + r
                w = 1.0 - plsc.load_gather(
                    mask_v, [lax.broadcast(row, (LANES,))])
                for d in range(DG):
                    sl = pl.ds(LANES * d, LANES)
                    emb_v[row, sl] = emb_v[row, sl] + avgs[d] * w
                return carry2

            lax.fori_loop(0, L, out_body, 0)

        def chunk_front(p, ids_off, idx_v, mask_v, sidx_v, base_row,
                        emb_v, sem_g, sem_s):
            build(ids_off, idx_v, mask_v)
            cnts = row_counts(mask_v)
            anyfix = cnts[0] != float(L)
            for c in cnts[1:]:
                anyfix = anyfix | (c != float(L))
            base = lax.broadcast(base_row, (LANES,))
            for j0 in range(0, TOK, LANES):
                sidx_v[j0 // UNIT, pl.ds(j0 % UNIT, LANES)] = (
                    pat_v[pl.ds(j0, LANES)] + base)
            cps = []
            for u in range(NU):
                sl = pl.ds(u * UNIT, UNIT)
                pl.when(p > 0)(
                    lambda u_=u, sl_=sl: pltpu.make_async_copy(
                        emb_v.at[sl_], out_hbm.at[pl.ds(0, UNIT)],
                        sem_s.at[u_]).wait())
                cps.append(pltpu.async_copy(
                    table_hbm.at[idx_v.at[sl]], emb_v.at[sl], sem_g.at[u]))
            return cnts, anyfix, cps

        def chunk_back(cnts, anyfix, cps, mask_v, emb_v, sidx_v, sem_s):
            def scatter(u):
                pltpu.async_copy(emb_v.at[pl.ds(u * UNIT, UNIT)],
                                 out_hbm.at[sidx_v.at[u]], sem_s.at[u])

            def fast():
                for u in range(NU):
                    cps[u].wait()
                    scatter(u)

            def slow():
                for u in range(NU):
                    cps[u].wait()
                for r in range(R):
                    pl.when(cnts[r] != float(L))(
                        lambda r_=r: fix_row(mask_v, emb_v, r_, cnts[r_]))
                for u in range(NU):
                    scatter(u)

            pl.when(jnp.logical_not(anyfix))(fast)
            pl.when(anyfix)(slow)

        def start_ids(p):
            base_tok = (wid * ROWS_W + p * 2 * R) * L
            pltpu.async_copy(ids_hbm.at[pl.ds(base_tok, 2 * TOK)],
                             ids_v, sem_i)

        def drain_final(emb_v, sem_s):
            for u in range(NU):
                pltpu.make_async_copy(
                    emb_v.at[pl.ds(u * UNIT, UNIT)],
                    out_hbm.at[pl.ds(0, UNIT)], sem_s.at[u]).wait()

        def pair_body(p, carry):
            base_row = wid * ROWS_W + p * 2 * R
            pltpu.make_async_copy(
                ids_hbm.at[pl.ds(0, 2 * TOK)], ids_v, sem_i).wait()
            st0 = chunk_front(p, 0, idx_v0, mask_v0, sidx_v0, base_row,
                              emb_v0, sem_g0, sem_s0)
            st1 = chunk_front(p, TOK, idx_v1, mask_v1, sidx_v1,
                              base_row + R, emb_v1, sem_g1, sem_s1)
            pl.when(p + 1 < NPAIR)(lambda: start_ids(p + 1))
            chunk_back(*st0, mask_v0, emb_v0, sidx_v0, sem_s0)
            chunk_back(*st1, mask_v1, emb_v1, sidx_v1, sem_s1)
            return carry

        for j0 in range(0, TOK, LANES):
            jv = iota + j0
            pat_v[pl.ds(j0, LANES)] = (
                lax.shift_right_logical(jv, RS) * B + (jv & (R - 1)))
        start_ids(0)
        lax.fori_loop(0, NPAIR, pair_body, 0)
        drain_final(emb_v0, sem_s0)
        drain_final(emb_v1, sem_s1)

    return embed_kernel


def kernel(input_ids, table):
    B, L = input_ids.shape
    V, D = table.shape
    k = _make_kernel(B, L, V, D)
    out = k(input_ids.reshape(-1).astype(jnp.int32), table)
    return jnp.swapaxes(out.reshape(L, B, D), 0, 1)


# speedup vs baseline: 11.7335x; 1.0004x over previous
import functools

import jax
import jax.numpy as jnp
from jax import lax
from jax.experimental import pallas as pl
from jax.experimental.pallas import tpu as pltpu
from jax.experimental.pallas import tpu_sc as plsc

NC = 2
NS = 16
LANES = 16
NW = NC * NS


@functools.lru_cache(maxsize=None)
def _make_kernel(B, L, V, D):
    UNK = V - 2
    PAD = V - 1
    DG = D // LANES
    ROWS_W = B // NW
    R = 8
    RS = R.bit_length() - 1
    TOK = R * L
    NCHUNK = ROWS_W // R
    NPAIR = NCHUNK // 2
    assert TOK % LANES == 0
    UNIT = 80
    NU = TOK // UNIT
    assert NU * UNIT == TOK and UNIT % LANES == 0 and UNIT <= 128

    mesh = plsc.VectorSubcoreMesh(
        core_axis_name="c", subcore_axis_name="s",
        num_cores=NC, num_subcores=NS)

    @functools.partial(
        pl.kernel,
        out_type=jax.ShapeDtypeStruct((B * L, D), jnp.float32),
        mesh=mesh,
        compiler_params=pltpu.CompilerParams(needs_layout_passes=False),
        scratch_types=[
            pltpu.VMEM((2 * TOK,), jnp.int32),
            pltpu.VMEM((TOK,), jnp.int32),
            pltpu.VMEM((TOK,), jnp.int32),
            pltpu.VMEM((TOK + 112,), jnp.float32),
            pltpu.VMEM((TOK + 112,), jnp.float32),
            pltpu.VMEM((TOK, D), jnp.float32),
            pltpu.VMEM((TOK, D), jnp.float32),
            pltpu.VMEM((TOK,), jnp.int32),
            pltpu.VMEM((NU, UNIT), jnp.int32),
            pltpu.VMEM((NU, UNIT), jnp.int32),
            pltpu.SemaphoreType.DMA((NU,)),
            pltpu.SemaphoreType.DMA((NU,)),
            pltpu.SemaphoreType.DMA((NU,)),
            pltpu.SemaphoreType.DMA((NU,)),
            pltpu.SemaphoreType.DMA,
        ],
    )
    def embed_kernel(ids_hbm, table_hbm, out_hbm, ids_v,
                     idx_v0, idx_v1, mask_v0, mask_v1, emb_v0, emb_v1,
                     pat_v, sidx_v0, sidx_v1,
                     sem_g0, sem_g1, sem_s0, sem_s1, sem_i):
        wid = lax.axis_index("s") * NC + lax.axis_index("c")
        iota = lax.iota(jnp.int32, LANES)

        def build(ids_off, idx_v, mask_v):
            for j0 in range(0, TOK, LANES):
                jv = iota + j0
                rv = jv & (R - 1)
                lv = lax.shift_right_logical(jv, RS)
                pos = rv * L + lv + ids_off
                idv = plsc.load_gather(ids_v, [pos])
                valid = (idv != UNK) & (idv != PAD)
                idx_v[pl.ds(j0, LANES)] = jnp.where(valid, idv, PAD)
                mask_v[pl.ds(j0, LANES)] = jnp.where(valid, 1.0, 0.0).astype(
                    jnp.float32)

        def row_counts(mask_v):
            cnts = []
            for r in range(R):
                g0 = plsc.load_gather(mask_v, [iota * R + r])
                g1 = plsc.load_gather(mask_v, [(iota + 16) * R + r])
                g2 = plsc.load_gather(mask_v, [(iota + 32) * R + r])
                g3 = plsc.load_gather(mask_v, [(iota + 48) * R + r])
                cnt_l = g0 + g1 + g2 + jnp.where(iota < L - 48, g3, 0.0)
                cnts.append(jnp.sum(cnt_l))
            return cnts

        def fix_row(mask_v, emb_v, r, cnt):
            def sum_body(t, accs):
                row = t * R + r
                return tuple(
                    accs[d] + emb_v[row, pl.ds(LANES * d, LANES)]
                    for d in range(DG))

            accs = lax.fori_loop(
                0, L, sum_body,
                tuple(jnp.zeros((LANES,), jnp.float32) for _ in range(DG)))
            denom = lax.broadcast(cnt, (LANES,)) + 1e-8
            avgs = tuple(a / denom for a in accs)

            def out_body(t, carry2):
                row = t * R + r
                w = 1.0 - plsc.load_gather(
                    mask_v, [lax.broadcast(row, (LANES,))])
                for d in range(DG):
                    sl = pl.ds(LANES * d, LANES)
                    emb_v[row, sl] = emb_v[row, sl] + avgs[d] * w
                return carry2

            lax.fori_loop(0, L, out_body, 0)

        def chunk_front(p, ids_off, idx_v, mask_v, sidx_v, base_row,
                        emb_v, sem_g, sem_s):
            build(ids_off, idx_v, mask_v)
            cnts = row_counts(mask_v)
            anyfix = cnts[0] != float(L)
            for c in cnts[1:]:
                anyfix = anyfix | (c != float(L))
            cps = []
            for u in range(NU):
                sl = pl.ds(u * UNIT, UNIT)
                pl.when(p > 0)(
                    lambda u_=u, sl_=sl: pltpu.make_async_copy(
                        emb_v.at[sl_], out_hbm.at[pl.ds(0, UNIT)],
                        sem_s.at[u_]).wait())
                cps.append(pltpu.async_copy(
                    table_hbm.at[idx_v.at[sl]], emb_v.at[sl], sem_g.at[u]))
            base = lax.broadcast(base_row, (LANES,))
            for j0 in range(0, TOK, LANES):
                sidx_v[j0 // UNIT, pl.ds(j0 % UNIT, LANES)] = (
                    pat_v[pl.ds(j0, LANES)] + base)
            return cnts, anyfix, cps

        def chunk_back(cnts, anyfix, cps, mask_v, emb_v, sidx_v, sem_s):
            def scatter(u):
                pltpu.async_copy(emb_v.at[pl.ds(u * UNIT, UNIT)],
                                 out_hbm.at[sidx_v.at[u]], sem_s.at[u])

            def fast():
                for u in range(NU):
                    cps[u].wait()
                    scatter(u)

            def slow():
                for u in range(NU):
                    cps[u].wait()
                for r in range(R):
                    pl.when(cnts[r] != float(L))(
                        lambda r_=r: fix_row(mask_v, emb_v, r_, cnts[r_]))
                for u in range(NU):
                    scatter(u)

            pl.when(jnp.logical_not(anyfix))(fast)
            pl.when(anyfix)(slow)

        def start_ids(p):
            base_tok = (wid * ROWS_W + p * 2 * R) * L
            pltpu.async_copy(ids_hbm.at[pl.ds(base_tok, 2 * TOK)],
                             ids_v, sem_i)

        def drain_final(emb_v, sem_s):
            for u in range(NU):
                pltpu.make_async_copy(
                    emb_v.at[pl.ds(u * UNIT, UNIT)],
                    out_hbm.at[pl.ds(0, UNIT)], sem_s.at[u]).wait()

        def pair_body(p, carry):
            base_row = wid * ROWS_W + p * 2 * R
            pltpu.make_async_copy(
                ids_hbm.at[pl.ds(0, 2 * TOK)], ids_v, sem_i).wait()
            st0 = chunk_front(p, 0, idx_v0, mask_v0, sidx_v0, base_row,
                              emb_v0, sem_g0, sem_s0)
            st1 = chunk_front(p, TOK, idx_v1, mask_v1, sidx_v1,
                              base_row + R, emb_v1, sem_g1, sem_s1)
            pl.when(p + 1 < NPAIR)(lambda: start_ids(p + 1))
            chunk_back(*st0, mask_v0, emb_v0, sidx_v0, sem_s0)
            chunk_back(*st1, mask_v1, emb_v1, sidx_v1, sem_s1)
            return carry

        for j0 in range(0, TOK, LANES):
            jv = iota + j0
            pat_v[pl.ds(j0, LANES)] = (
                lax.shift_right_logical(jv, RS) * B + (jv & (R - 1)))
        start_ids(0)
        lax.fori_loop(0, NPAIR, pair_body, 0)
        drain_final(emb_v0, sem_s0)
        drain_final(emb_v1, sem_s1)

    return embed_kernel


def kernel(input_ids, table):
    B, L = input_ids.shape
    V, D = table.shape
    k = _make_kernel(B, L, V, D)
    out = k(input_ids.reshape(-1).astype(jnp.int32), table)
    return jnp.swapaxes(out.reshape(L, B, D), 0, 1)


# speedup vs baseline: 11.7579x; 1.0021x over previous
import functools

import jax
import jax.numpy as jnp
from jax import lax
from jax.experimental import pallas as pl
from jax.experimental.pallas import tpu as pltpu
from jax.experimental.pallas import tpu_sc as plsc

NC = 2
NS = 16
LANES = 16
NW = NC * NS


@functools.lru_cache(maxsize=None)
def _make_kernel(B, L, V, D):
    UNK = V - 2
    PAD = V - 1
    DG = D // LANES
    ROWS_W = B // NW
    R = 8
    RS = R.bit_length() - 1
    TOK = R * L
    NCHUNK = ROWS_W // R
    NPAIR = NCHUNK // 2
    assert TOK % LANES == 0
    UNIT = 80
    NU = TOK // UNIT
    assert NU * UNIT == TOK and UNIT % LANES == 0 and UNIT <= 128

    mesh = plsc.VectorSubcoreMesh(
        core_axis_name="c", subcore_axis_name="s",
        num_cores=NC, num_subcores=NS)

    @functools.partial(
        pl.kernel,
        out_type=jax.ShapeDtypeStruct((B * L, D), jnp.float32),
        mesh=mesh,
        compiler_params=pltpu.CompilerParams(needs_layout_passes=False),
        scratch_types=[
            pltpu.VMEM((2 * TOK,), jnp.int32),
            pltpu.VMEM((TOK,), jnp.int32),
            pltpu.VMEM((TOK,), jnp.int32),
            pltpu.VMEM((TOK + 112,), jnp.float32),
            pltpu.VMEM((TOK + 112,), jnp.float32),
            pltpu.VMEM((TOK, D), jnp.float32),
            pltpu.VMEM((TOK, D), jnp.float32),
            pltpu.VMEM((TOK,), jnp.int32),
            pltpu.VMEM((NU, UNIT), jnp.int32),
            pltpu.VMEM((NU, UNIT), jnp.int32),
            pltpu.SemaphoreType.DMA((NU,)),
            pltpu.SemaphoreType.DMA((NU,)),
            pltpu.SemaphoreType.DMA((NU,)),
            pltpu.SemaphoreType.DMA((NU,)),
            pltpu.SemaphoreType.DMA,
        ],
    )
    def embed_kernel(ids_hbm, table_hbm, out_hbm, ids_v,
                     idx_v0, idx_v1, mask_v0, mask_v1, emb_v0, emb_v1,
                     pat_v, sidx_v0, sidx_v1,
                     sem_g0, sem_g1, sem_s0, sem_s1, sem_i):
        wid = lax.axis_index("s") * NC + lax.axis_index("c")
        iota = lax.iota(jnp.int32, LANES)

        def build(ids_off, idx_v, mask_v):
            allvalid = None
            for j0 in range(0, TOK, LANES):
                jv = iota + j0
                rv = jv & (R - 1)
                lv = lax.shift_right_logical(jv, RS)
                pos = rv * L + lv + ids_off
                idv = plsc.load_gather(ids_v, [pos])
                valid = (idv != UNK) & (idv != PAD)
                allvalid = valid if allvalid is None else (allvalid & valid)
                idx_v[pl.ds(j0, LANES)] = jnp.where(valid, idv, PAD)
                mask_v[pl.ds(j0, LANES)] = jnp.where(valid, 1.0, 0.0).astype(
                    jnp.float32)
            return allvalid

        def row_counts(mask_v):
            cnts = []
            for r in range(R):
                g0 = plsc.load_gather(mask_v, [iota * R + r])
                g1 = plsc.load_gather(mask_v, [(iota + 16) * R + r])
                g2 = plsc.load_gather(mask_v, [(iota + 32) * R + r])
                g3 = plsc.load_gather(mask_v, [(iota + 48) * R + r])
                cnt_l = g0 + g1 + g2 + jnp.where(iota < L - 48, g3, 0.0)
                cnts.append(jnp.sum(cnt_l))
            return cnts

        def fix_row(mask_v, emb_v, r, cnt):
            def sum_body(t, accs):
                row = t * R + r
                return tuple(
                    accs[d] + emb_v[row, pl.ds(LANES * d, LANES)]
                    for d in range(DG))

            accs = lax.fori_loop(
                0, L, sum_body,
                tuple(jnp.zeros((LANES,), jnp.float32) for _ in range(DG)))
            denom = lax.broadcast(cnt, (LANES,)) + 1e-8
            avgs = tuple(a / denom for a in accs)

            def out_body(t, carry2):
                row = t * R + r
                w = 1.0 - plsc.load_gather(
                    mask_v, [lax.broadcast(row, (LANES,))])
                for d in range(DG):
                    sl = pl.ds(LANES * d, LANES)
                    emb_v[row, sl] = emb_v[row, sl] + avgs[d] * w
                return carry2

            lax.fori_loop(0, L, out_body, 0)

        def chunk_front(p, ids_off, idx_v, mask_v, sidx_v, base_row,
                        emb_v, sem_g, sem_s):
            allvalid = build(ids_off, idx_v, mask_v)
            anyfix = jnp.logical_not(jnp.all(allvalid))
            cps = []
            for u in range(NU):
                sl = pl.ds(u * UNIT, UNIT)
                pl.when(p > 0)(
                    lambda u_=u, sl_=sl: pltpu.make_async_copy(
                        emb_v.at[sl_], out_hbm.at[pl.ds(0, UNIT)],
                        sem_s.at[u_]).wait())
                cps.append(pltpu.async_copy(
                    table_hbm.at[idx_v.at[sl]], emb_v.at[sl], sem_g.at[u]))
            base = lax.broadcast(base_row, (LANES,))
            for j0 in range(0, TOK, LANES):
                sidx_v[j0 // UNIT, pl.ds(j0 % UNIT, LANES)] = (
                    pat_v[pl.ds(j0, LANES)] + base)
            return anyfix, cps

        def chunk_back(anyfix, cps, mask_v, emb_v, sidx_v, sem_s):
            def scatter(u):
                pltpu.async_copy(emb_v.at[pl.ds(u * UNIT, UNIT)],
                                 out_hbm.at[sidx_v.at[u]], sem_s.at[u])

            def fast():
                for u in range(NU):
                    cps[u].wait()
                    scatter(u)

            def slow():
                for u in range(NU):
                    cps[u].wait()
                cnts = row_counts(mask_v)
                for r in range(R):
                    pl.when(cnts[r] != float(L))(
                        lambda r_=r: fix_row(mask_v, emb_v, r_, cnts[r_]))
                for u in range(NU):
                    scatter(u)

            pl.when(jnp.logical_not(anyfix))(fast)
            pl.when(anyfix)(slow)

        def start_ids(p):
            base_tok = (wid * ROWS_W + p * 2 * R) * L
            pltpu.async_copy(ids_hbm.at[pl.ds(base_tok, 2 * TOK)],
                             ids_v, sem_i)

        def drain_final(emb_v, sem_s):
            for u in range(NU):
                pltpu.make_async_copy(
                    emb_v.at[pl.ds(u * UNIT, UNIT)],
                    out_hbm.at[pl.ds(0, UNIT)], sem_s.at[u]).wait()

        def pair_body(p, carry):
            base_row = wid * ROWS_W + p * 2 * R
            pltpu.make_async_copy(
                ids_hbm.at[pl.ds(0, 2 * TOK)], ids_v, sem_i).wait()
            st0 = chunk_front(p, 0, idx_v0, mask_v0, sidx_v0, base_row,
                              emb_v0, sem_g0, sem_s0)
            st1 = chunk_front(p, TOK, idx_v1, mask_v1, sidx_v1,
                              base_row + R, emb_v1, sem_g1, sem_s1)
            pl.when(p + 1 < NPAIR)(lambda: start_ids(p + 1))
            chunk_back(*st0, mask_v0, emb_v0, sidx_v0, sem_s0)
            chunk_back(*st1, mask_v1, emb_v1, sidx_v1, sem_s1)
            return carry

        for j0 in range(0, TOK, LANES):
            jv = iota + j0
            pat_v[pl.ds(j0, LANES)] = (
                lax.shift_right_logical(jv, RS) * B + (jv & (R - 1)))
        start_ids(0)
        lax.fori_loop(0, NPAIR, pair_body, 0)
        drain_final(emb_v0, sem_s0)
        drain_final(emb_v1, sem_s1)

    return embed_kernel


def kernel(input_ids, table):
    B, L = input_ids.shape
    V, D = table.shape
    k = _make_kernel(B, L, V, D)
    out = k(input_ids.reshape(-1).astype(jnp.int32), table)
    return jnp.swapaxes(out.reshape(L, B, D), 0, 1)
